# baseline (speedup 1.0000x reference)
import functools

import jax
import jax.numpy as jnp
from jax.experimental import pallas as pl
from jax.experimental.pallas import tpu as pltpu

_B = 64
_V = 4096
_C = 4096
_ITERS = 15
_TILE = 512
_KT = _C // _TILE

_DIMS_NN = (((1,), (0,)), ((), ()))


def _bp_body(syn_ref, ht_ref, h_ref, llrf_ref, llrt_ref,
             wcv_ref, wvc_ref, damp_ref,
             out_ref, x_ref, xb_ref, cb_ref):
    i = pl.program_id(0)
    p = pl.program_id(1)
    t = pl.program_id(2)
    wcv = wcv_ref[0, 0]
    wvc = wvc_ref[0, 0]
    damp = damp_ref[0, 0]
    dst = pl.ds(t * _TILE, _TILE)

    @pl.when((i == 0) & (p == 0) & (t == 0))
    def _():
        llr = llrf_ref[...]
        xb_ref[...] = llr.astype(jnp.bfloat16)

    @pl.when(p == 0)
    def _():
        v_to_c = wvc * jax.lax.dot_general(
            xb_ref[...], ht_ref[...], _DIMS_NN,
            preferred_element_type=jnp.float32)
        s_sign = 1.0 - 2.0 * syn_ref[...].astype(jnp.float32)
        c_msg = s_sign * jnp.tanh(v_to_c * 0.5)
        cb_ref[:, dst] = c_msg.astype(jnp.bfloat16)

    @pl.when(p == 1)
    def _():
        c_to_v = wcv * jax.lax.dot_general(
            cb_ref[...], h_ref[...], _DIMS_NN,
            preferred_element_type=jnp.float32)
        x_t = jnp.where(i == 0, llrt_ref[...], x_ref[:, dst])
        x_new = damp * x_t + (1.0 - damp) * (llrt_ref[...] + c_to_v)
        x_ref[:, dst] = x_new
        xb_ref[:, dst] = x_new.astype(jnp.bfloat16)

        @pl.when(i == _ITERS - 1)
        def _():
            out_ref[...] = jax.nn.sigmoid(-x_new)


@functools.partial(jax.jit, static_argnames=())
def kernel(syndrome, parity_matrix, channel_llrs, w_cv, w_vc, damping):
    h_bf = parity_matrix.astype(jnp.bfloat16)
    ht_bf = h_bf.T
    wcv = jnp.reshape(w_cv.astype(jnp.float32), (1, 1))
    wvc = jnp.reshape(w_vc.astype(jnp.float32), (1, 1))
    damp = jnp.reshape(damping.astype(jnp.float32), (1, 1))
    out = pl.pallas_call(
        _bp_body,
        grid=(_ITERS, 2, _KT),
        out_shape=jax.ShapeDtypeStruct((_B, _V), jnp.float32),
        in_specs=[
            pl.BlockSpec((_B, _TILE), lambda i, p, t: (0, t)),
            pl.BlockSpec((_V, _TILE), lambda i, p, t: (0, jnp.where(p == 0, t, _KT - 1))),
            pl.BlockSpec((_C, _TILE), lambda i, p, t: (0, jnp.where(p == 1, t, 0))),
            pl.BlockSpec((_B, _V), lambda i, p, t: (0, 0)),
            pl.BlockSpec((_B, _TILE), lambda i, p, t: (0, t)),
            pl.BlockSpec((1, 1), lambda i, p, t: (0, 0), memory_space=pltpu.SMEM),
            pl.BlockSpec((1, 1), lambda i, p, t: (0, 0), memory_space=pltpu.SMEM),
            pl.BlockSpec((1, 1), lambda i, p, t: (0, 0), memory_space=pltpu.SMEM),
        ],
        out_specs=pl.BlockSpec((_B, _TILE), lambda i, p, t: (0, t)),
        scratch_shapes=[
            pltpu.VMEM((_B, _V), jnp.float32),
            pltpu.VMEM((_B, _V), jnp.bfloat16),
            pltpu.VMEM((_B, _C), jnp.bfloat16),
        ],
    )(syndrome, ht_bf, h_bf, channel_llrs, channel_llrs, wcv, wvc, damp)
    return out


# speedup vs baseline: 1.4589x; 1.4589x over previous
import functools

import jax
import jax.numpy as jnp
from jax.experimental import pallas as pl
from jax.experimental.pallas import tpu as pltpu

_B = 64
_V = 4096
_C = 4096
_ITERS = 15
_TILE = 512
_KT = _C // _TILE

_DIMS_NN = (((1,), (0,)), ((), ()))


def _bp_body(syn_ref, ht_ref, h_ref, llrf_ref, llrt_ref,
             wcv_ref, wvc_ref, damp_ref,
             out_ref, x_ref, xb_ref, cb_ref):
    i = pl.program_id(0)
    p = pl.program_id(1)
    t = pl.program_id(2)
    wcv = wcv_ref[0, 0]
    wvc = wvc_ref[0, 0]
    damp = damp_ref[0, 0]
    dst = pl.ds(t * _TILE, _TILE)

    @pl.when((i == 0) & (p == 0) & (t == 0))
    def _():
        llr = llrf_ref[...]
        xb_ref[...] = llr.astype(jnp.bfloat16)

    @pl.when(p == 0)
    def _():
        v_to_c = wvc * jax.lax.dot_general(
            xb_ref[...], ht_ref[...], _DIMS_NN,
            preferred_element_type=jnp.float32)
        s_sign = 1.0 - 2.0 * syn_ref[...].astype(jnp.float32)
        c_msg = s_sign * jnp.tanh(v_to_c * 0.5)
        cb_ref[:, dst] = c_msg.astype(jnp.bfloat16)

    @pl.when(p == 1)
    def _():
        c_to_v = wcv * jax.lax.dot_general(
            cb_ref[...], h_ref[...], _DIMS_NN,
            preferred_element_type=jnp.float32)
        x_t = jnp.where(i == 0, llrt_ref[...], x_ref[:, dst])
        x_new = damp * x_t + (1.0 - damp) * (llrt_ref[...] + c_to_v)
        x_ref[:, dst] = x_new
        xb_ref[:, dst] = x_new.astype(jnp.bfloat16)

        @pl.when(i == _ITERS - 1)
        def _():
            out_ref[...] = jax.nn.sigmoid(-x_new)


@functools.partial(jax.jit, static_argnames=())
def kernel(syndrome, parity_matrix, channel_llrs, w_cv, w_vc, damping):
    h_bf = parity_matrix.astype(jnp.float8_e4m3fn)
    ht_bf = h_bf.T
    wcv = jnp.reshape(w_cv.astype(jnp.float32), (1, 1))
    wvc = jnp.reshape(w_vc.astype(jnp.float32), (1, 1))
    damp = jnp.reshape(damping.astype(jnp.float32), (1, 1))
    out = pl.pallas_call(
        _bp_body,
        grid=(_ITERS, 2, _KT),
        out_shape=jax.ShapeDtypeStruct((_B, _V), jnp.float32),
        in_specs=[
            pl.BlockSpec((_B, _TILE), lambda i, p, t: (0, t)),
            pl.BlockSpec((_V, _TILE), lambda i, p, t: (0, jnp.where(p == 0, t, _KT - 1))),
            pl.BlockSpec((_C, _TILE), lambda i, p, t: (0, jnp.where(p == 1, t, 0))),
            pl.BlockSpec((_B, _V), lambda i, p, t: (0, 0)),
            pl.BlockSpec((_B, _TILE), lambda i, p, t: (0, t)),
            pl.BlockSpec((1, 1), lambda i, p, t: (0, 0), memory_space=pltpu.SMEM),
            pl.BlockSpec((1, 1), lambda i, p, t: (0, 0), memory_space=pltpu.SMEM),
            pl.BlockSpec((1, 1), lambda i, p, t: (0, 0), memory_space=pltpu.SMEM),
        ],
        out_specs=pl.BlockSpec((_B, _TILE), lambda i, p, t: (0, t)),
        scratch_shapes=[
            pltpu.VMEM((_B, _V), jnp.float32),
            pltpu.VMEM((_B, _V), jnp.bfloat16),
            pltpu.VMEM((_B, _C), jnp.bfloat16),
        ],
    )(syndrome, ht_bf, h_bf, channel_llrs, channel_llrs, wcv, wvc, damp)
    return out


# speedup vs baseline: 1.8897x; 1.2952x over previous
import functools

import jax
import jax.numpy as jnp
from jax.experimental import pallas as pl
from jax.experimental.pallas import tpu as pltpu

_B = 64
_V = 4096
_C = 4096
_ITERS = 15
_TILE = 512
_KT = _C // _TILE

_DIMS_NN = (((1,), (0,)), ((), ()))


def _bp_body(syn_ref, ht_ref, h_ref, llrf_ref, llrt_ref,
             wcv_ref, wvc_ref, damp_ref,
             out_ref, x_ref, xb_ref, cb_ref):
    i = pl.program_id(0)
    p = pl.program_id(1)
    t = pl.program_id(2)
    wcv = wcv_ref[0, 0]
    wvc = wvc_ref[0, 0]
    damp = damp_ref[0, 0]
    dst = pl.ds(t * _TILE, _TILE)

    @pl.when((i == 0) & (p == 0) & (t == 0))
    def _():
        llr = llrf_ref[...]
        xb_ref[...] = llr.astype(jnp.bfloat16)

    @pl.when(p == 0)
    def _():
        v_to_c = wvc * jax.lax.dot_general(
            xb_ref[...], ht_ref[:, dst], _DIMS_NN,
            preferred_element_type=jnp.float32)
        s_sign = 1.0 - 2.0 * syn_ref[...].astype(jnp.float32)
        c_msg = s_sign * jnp.tanh(v_to_c * 0.5)
        cb_ref[:, dst] = c_msg.astype(jnp.bfloat16)

    @pl.when(p == 1)
    def _():
        c_to_v = wcv * jax.lax.dot_general(
            cb_ref[...], h_ref[:, dst], _DIMS_NN,
            preferred_element_type=jnp.float32)
        x_t = jnp.where(i == 0, llrt_ref[...], x_ref[:, dst])
        x_new = damp * x_t + (1.0 - damp) * (llrt_ref[...] + c_to_v)
        x_ref[:, dst] = x_new
        xb_ref[:, dst] = x_new.astype(jnp.bfloat16)

        @pl.when(i == _ITERS - 1)
        def _():
            out_ref[...] = jax.nn.sigmoid(-x_new)


@functools.partial(jax.jit, static_argnames=())
def kernel(syndrome, parity_matrix, channel_llrs, w_cv, w_vc, damping):
    h_bf = parity_matrix.astype(jnp.float8_e4m3fn)
    ht_bf = h_bf.T
    wcv = jnp.reshape(w_cv.astype(jnp.float32), (1, 1))
    wvc = jnp.reshape(w_vc.astype(jnp.float32), (1, 1))
    damp = jnp.reshape(damping.astype(jnp.float32), (1, 1))
    out = pl.pallas_call(
        _bp_body,
        grid=(_ITERS, 2, _KT),
        out_shape=jax.ShapeDtypeStruct((_B, _V), jnp.float32),
        in_specs=[
            pl.BlockSpec((_B, _TILE), lambda i, p, t: (0, t)),
            pl.BlockSpec((_V, _C), lambda i, p, t: (0, 0)),
            pl.BlockSpec((_C, _V), lambda i, p, t: (0, 0)),
            pl.BlockSpec((_B, _V), lambda i, p, t: (0, 0)),
            pl.BlockSpec((_B, _TILE), lambda i, p, t: (0, t)),
            pl.BlockSpec((1, 1), lambda i, p, t: (0, 0), memory_space=pltpu.SMEM),
            pl.BlockSpec((1, 1), lambda i, p, t: (0, 0), memory_space=pltpu.SMEM),
            pl.BlockSpec((1, 1), lambda i, p, t: (0, 0), memory_space=pltpu.SMEM),
        ],
        out_specs=pl.BlockSpec((_B, _TILE), lambda i, p, t: (0, t)),
        scratch_shapes=[
            pltpu.VMEM((_B, _V), jnp.float32),
            pltpu.VMEM((_B, _V), jnp.bfloat16),
            pltpu.VMEM((_B, _C), jnp.bfloat16),
        ],
    )(syndrome, ht_bf, h_bf, channel_llrs, channel_llrs, wcv, wvc, damp)
    return out


# speedup vs baseline: 2.4987x; 1.3223x over previous
import functools

import jax
import jax.numpy as jnp
from jax.experimental import pallas as pl
from jax.experimental.pallas import tpu as pltpu

_B = 64
_V = 4096
_C = 4096
_ITERS = 15
_TILE = 512
_KT = _C // _TILE

_DIMS_NN = (((1,), (0,)), ((), ()))


def _bp_body(syn_ref, ht_ref, h_ref, llr_ref,
             wcv_ref, wvc_ref, damp_ref,
             out_ref, x_ref, xb_ref, cb_ref):
    i = pl.program_id(0)
    wcv = wcv_ref[0, 0]
    wvc = wvc_ref[0, 0]
    damp = damp_ref[0, 0]

    @pl.when(i == 0)
    def _():
        xb_ref[...] = llr_ref[...].astype(jnp.bfloat16)

    def ph0(t, carry):
        dst = pl.ds(t * _TILE, _TILE)
        v_to_c = wvc * jax.lax.dot_general(
            xb_ref[...], ht_ref[:, dst], _DIMS_NN,
            preferred_element_type=jnp.float32)
        s_sign = 1.0 - 2.0 * syn_ref[:, dst].astype(jnp.float32)
        c_msg = s_sign * jnp.tanh(v_to_c * 0.5)
        cb_ref[:, dst] = c_msg.astype(jnp.bfloat16)
        return carry

    jax.lax.fori_loop(0, _KT, ph0, 0)

    def ph1(t, carry):
        dst = pl.ds(t * _TILE, _TILE)
        c_to_v = wcv * jax.lax.dot_general(
            cb_ref[...], h_ref[:, dst], _DIMS_NN,
            preferred_element_type=jnp.float32)
        llr_t = llr_ref[:, dst]
        x_t = jnp.where(i == 0, llr_t, x_ref[:, dst])
        x_new = damp * x_t + (1.0 - damp) * (llr_t + c_to_v)
        x_ref[:, dst] = x_new
        xb_ref[:, dst] = x_new.astype(jnp.bfloat16)

        @pl.when(i == _ITERS - 1)
        def _():
            out_ref[:, dst] = jax.nn.sigmoid(-x_new)

        return carry

    jax.lax.fori_loop(0, _KT, ph1, 0)


@functools.partial(jax.jit, static_argnames=())
def kernel(syndrome, parity_matrix, channel_llrs, w_cv, w_vc, damping):
    h_f8 = parity_matrix.astype(jnp.float8_e4m3fn)
    ht_f8 = h_f8.T
    wcv = jnp.reshape(w_cv.astype(jnp.float32), (1, 1))
    wvc = jnp.reshape(w_vc.astype(jnp.float32), (1, 1))
    damp = jnp.reshape(damping.astype(jnp.float32), (1, 1))
    out = pl.pallas_call(
        _bp_body,
        grid=(_ITERS,),
        out_shape=jax.ShapeDtypeStruct((_B, _V), jnp.float32),
        in_specs=[
            pl.BlockSpec((_B, _C), lambda i: (0, 0)),
            pl.BlockSpec((_V, _C), lambda i: (0, 0)),
            pl.BlockSpec((_C, _V), lambda i: (0, 0)),
            pl.BlockSpec((_B, _V), lambda i: (0, 0)),
            pl.BlockSpec((1, 1), lambda i: (0, 0), memory_space=pltpu.SMEM),
            pl.BlockSpec((1, 1), lambda i: (0, 0), memory_space=pltpu.SMEM),
            pl.BlockSpec((1, 1), lambda i: (0, 0), memory_space=pltpu.SMEM),
        ],
        out_specs=pl.BlockSpec((_B, _V), lambda i: (0, 0)),
        scratch_shapes=[
            pltpu.VMEM((_B, _V), jnp.float32),
            pltpu.VMEM((_B, _V), jnp.bfloat16),
            pltpu.VMEM((_B, _C), jnp.bfloat16),
        ],
    )(syndrome, ht_f8, h_f8, channel_llrs, wcv, wvc, damp)
    return out


# speedup vs baseline: 2.7042x; 1.0823x over previous
import functools

import jax
import jax.numpy as jnp
from jax.experimental import pallas as pl
from jax.experimental.pallas import tpu as pltpu

_B = 64
_V = 4096
_C = 4096
_ITERS = 15
_TILE = 512
_KT = _C // _TILE

_DIMS_NN = (((1,), (0,)), ((), ()))


def _bp_body(syn_ref, ht_ref, h_ref, llr_ref,
             wcv_ref, wvc_ref, damp_ref,
             out_ref, x_ref, xb_ref, cb_ref):
    i = pl.program_id(0)
    wcv = wcv_ref[0, 0]
    wvc = wvc_ref[0, 0]
    damp = damp_ref[0, 0]

    @pl.when(i == 0)
    def _():
        xb_ref[...] = llr_ref[...].astype(jnp.bfloat16)

    def ph0(t, carry):
        dst = pl.ds(t * _TILE, _TILE)
        v_to_c = wvc * jax.lax.dot_general(
            xb_ref[...], ht_ref[:, dst], _DIMS_NN,
            preferred_element_type=jnp.float32)
        s_sign = 1.0 - 2.0 * syn_ref[:, dst].astype(jnp.float32)
        c_msg = s_sign * jnp.tanh(v_to_c * 0.5)
        cb_ref[:, dst] = c_msg.astype(jnp.bfloat16)
        return carry

    jax.lax.fori_loop(0, _KT, ph0, 0, unroll=2)

    def ph1(t, carry):
        dst = pl.ds(t * _TILE, _TILE)
        c_to_v = wcv * jax.lax.dot_general(
            cb_ref[...], h_ref[:, dst], _DIMS_NN,
            preferred_element_type=jnp.float32)
        llr_t = llr_ref[:, dst]
        x_t = jnp.where(i == 0, llr_t, x_ref[:, dst])
        x_new = damp * x_t + (1.0 - damp) * (llr_t + c_to_v)
        x_ref[:, dst] = x_new
        xb_ref[:, dst] = x_new.astype(jnp.bfloat16)

        @pl.when(i == _ITERS - 1)
        def _():
            out_ref[:, dst] = jax.nn.sigmoid(-x_new)

        return carry

    jax.lax.fori_loop(0, _KT, ph1, 0, unroll=2)


@functools.partial(jax.jit, static_argnames=())
def kernel(syndrome, parity_matrix, channel_llrs, w_cv, w_vc, damping):
    h_f8 = parity_matrix.astype(jnp.float8_e4m3fn)
    ht_f8 = h_f8.T
    wcv = jnp.reshape(w_cv.astype(jnp.float32), (1, 1))
    wvc = jnp.reshape(w_vc.astype(jnp.float32), (1, 1))
    damp = jnp.reshape(damping.astype(jnp.float32), (1, 1))
    out = pl.pallas_call(
        _bp_body,
        grid=(_ITERS,),
        out_shape=jax.ShapeDtypeStruct((_B, _V), jnp.float32),
        in_specs=[
            pl.BlockSpec((_B, _C), lambda i: (0, 0)),
            pl.BlockSpec((_V, _C), lambda i: (0, 0)),
            pl.BlockSpec((_C, _V), lambda i: (0, 0)),
            pl.BlockSpec((_B, _V), lambda i: (0, 0)),
            pl.BlockSpec((1, 1), lambda i: (0, 0), memory_space=pltpu.SMEM),
            pl.BlockSpec((1, 1), lambda i: (0, 0), memory_space=pltpu.SMEM),
            pl.BlockSpec((1, 1), lambda i: (0, 0), memory_space=pltpu.SMEM),
        ],
        out_specs=pl.BlockSpec((_B, _V), lambda i: (0, 0)),
        scratch_shapes=[
            pltpu.VMEM((_B, _V), jnp.float32),
            pltpu.VMEM((_B, _V), jnp.bfloat16),
            pltpu.VMEM((_B, _C), jnp.bfloat16),
        ],
    )(syndrome, ht_f8, h_f8, channel_llrs, wcv, wvc, damp)
    return out


# speedup vs baseline: 2.7949x; 1.0335x over previous
import functools

import jax
import jax.numpy as jnp
from jax.experimental import pallas as pl
from jax.experimental.pallas import tpu as pltpu

_B = 64
_V = 4096
_C = 4096
_ITERS = 15
_TILE = 512
_KT = _C // _TILE

_DIMS_NN = (((1,), (0,)), ((), ()))


def _bp_body(syn_ref, ht_ref, h_ref, llr_ref,
             wcv_ref, wvc_ref, damp_ref,
             out_ref, x_ref, xb_ref, cb_ref):
    i = pl.program_id(0)
    wcv = wcv_ref[0, 0]
    wvc = wvc_ref[0, 0]
    damp = damp_ref[0, 0]

    @pl.when(i == 0)
    def _():
        xb_ref[...] = llr_ref[...].astype(jnp.bfloat16)

    def ph0(t, carry):
        dst = pl.ds(t * _TILE, _TILE)
        v_to_c = wvc * jax.lax.dot_general(
            xb_ref[...], ht_ref[:, dst], _DIMS_NN,
            preferred_element_type=jnp.float32)
        s_sign = 1.0 - 2.0 * syn_ref[:, dst].astype(jnp.float32)
        c_msg = s_sign * jnp.tanh(v_to_c * 0.5)
        cb_ref[:, dst] = c_msg.astype(jnp.bfloat16)
        return carry

    jax.lax.fori_loop(0, _KT, ph0, 0, unroll=4)

    def ph1(t, carry):
        dst = pl.ds(t * _TILE, _TILE)
        c_to_v = wcv * jax.lax.dot_general(
            cb_ref[...], h_ref[:, dst], _DIMS_NN,
            preferred_element_type=jnp.float32)
        llr_t = llr_ref[:, dst]
        x_t = jnp.where(i == 0, llr_t, x_ref[:, dst])
        x_new = damp * x_t + (1.0 - damp) * (llr_t + c_to_v)
        x_ref[:, dst] = x_new
        xb_ref[:, dst] = x_new.astype(jnp.bfloat16)

        @pl.when(i == _ITERS - 1)
        def _():
            out_ref[:, dst] = jax.nn.sigmoid(-x_new)

        return carry

    jax.lax.fori_loop(0, _KT, ph1, 0, unroll=4)


@functools.partial(jax.jit, static_argnames=())
def kernel(syndrome, parity_matrix, channel_llrs, w_cv, w_vc, damping):
    h_f8 = parity_matrix.astype(jnp.float8_e4m3fn)
    ht_f8 = h_f8.T
    wcv = jnp.reshape(w_cv.astype(jnp.float32), (1, 1))
    wvc = jnp.reshape(w_vc.astype(jnp.float32), (1, 1))
    damp = jnp.reshape(damping.astype(jnp.float32), (1, 1))
    out = pl.pallas_call(
        _bp_body,
        grid=(_ITERS,),
        out_shape=jax.ShapeDtypeStruct((_B, _V), jnp.float32),
        in_specs=[
            pl.BlockSpec((_B, _C), lambda i: (0, 0)),
            pl.BlockSpec((_V, _C), lambda i: (0, 0)),
            pl.BlockSpec((_C, _V), lambda i: (0, 0)),
            pl.BlockSpec((_B, _V), lambda i: (0, 0)),
            pl.BlockSpec((1, 1), lambda i: (0, 0), memory_space=pltpu.SMEM),
            pl.BlockSpec((1, 1), lambda i: (0, 0), memory_space=pltpu.SMEM),
            pl.BlockSpec((1, 1), lambda i: (0, 0), memory_space=pltpu.SMEM),
        ],
        out_specs=pl.BlockSpec((_B, _V), lambda i: (0, 0)),
        scratch_shapes=[
            pltpu.VMEM((_B, _V), jnp.float32),
            pltpu.VMEM((_B, _V), jnp.bfloat16),
            pltpu.VMEM((_B, _C), jnp.bfloat16),
        ],
    )(syndrome, ht_f8, h_f8, channel_llrs, wcv, wvc, damp)
    return out


# speedup vs baseline: 2.9032x; 1.0388x over previous
import functools

import jax
import jax.numpy as jnp
from jax.experimental import pallas as pl
from jax.experimental.pallas import tpu as pltpu

_B = 64
_V = 4096
_C = 4096
_ITERS = 15
_TILE = 512
_KT = _C // _TILE

_DIMS_NN = (((1,), (0,)), ((), ()))


def _bp_body(syn_ref, ht_ref, h_ref, llr_ref,
             wcv_ref, wvc_ref, damp_ref,
             out_ref, x_ref, xb_ref, cb_ref):
    i = pl.program_id(0)
    wcv = wcv_ref[0, 0]
    wvc = wvc_ref[0, 0]
    damp = damp_ref[0, 0]

    @pl.when(i == 0)
    def _():
        xb_ref[...] = llr_ref[...].astype(jnp.bfloat16)

    def ph0(t, carry):
        dst = pl.ds(t * _TILE, _TILE)
        v_to_c = wvc * jax.lax.dot_general(
            xb_ref[...], ht_ref[:, dst], _DIMS_NN,
            preferred_element_type=jnp.float32)
        s_sign = 1.0 - 2.0 * syn_ref[:, dst].astype(jnp.float32)
        c_msg = s_sign * jnp.tanh(v_to_c * 0.5)
        cb_ref[:, dst] = c_msg.astype(jnp.bfloat16)
        return carry

    jax.lax.fori_loop(0, _KT, ph0, 0, unroll=8)

    def ph1(t, carry):
        dst = pl.ds(t * _TILE, _TILE)
        c_to_v = wcv * jax.lax.dot_general(
            cb_ref[...], h_ref[:, dst], _DIMS_NN,
            preferred_element_type=jnp.float32)
        llr_t = llr_ref[:, dst]
        x_t = jnp.where(i == 0, llr_t, x_ref[:, dst])
        x_new = damp * x_t + (1.0 - damp) * (llr_t + c_to_v)
        x_ref[:, dst] = x_new
        xb_ref[:, dst] = x_new.astype(jnp.bfloat16)

        @pl.when(i == _ITERS - 1)
        def _():
            out_ref[:, dst] = jax.nn.sigmoid(-x_new)

        return carry

    jax.lax.fori_loop(0, _KT, ph1, 0, unroll=8)


@functools.partial(jax.jit, static_argnames=())
def kernel(syndrome, parity_matrix, channel_llrs, w_cv, w_vc, damping):
    h_f8 = parity_matrix.astype(jnp.float8_e4m3fn)
    ht_f8 = h_f8.T
    wcv = jnp.reshape(w_cv.astype(jnp.float32), (1, 1))
    wvc = jnp.reshape(w_vc.astype(jnp.float32), (1, 1))
    damp = jnp.reshape(damping.astype(jnp.float32), (1, 1))
    out = pl.pallas_call(
        _bp_body,
        grid=(_ITERS,),
        out_shape=jax.ShapeDtypeStruct((_B, _V), jnp.float32),
        in_specs=[
            pl.BlockSpec((_B, _C), lambda i: (0, 0)),
            pl.BlockSpec((_V, _C), lambda i: (0, 0)),
            pl.BlockSpec((_C, _V), lambda i: (0, 0)),
            pl.BlockSpec((_B, _V), lambda i: (0, 0)),
            pl.BlockSpec((1, 1), lambda i: (0, 0), memory_space=pltpu.SMEM),
            pl.BlockSpec((1, 1), lambda i: (0, 0), memory_space=pltpu.SMEM),
            pl.BlockSpec((1, 1), lambda i: (0, 0), memory_space=pltpu.SMEM),
        ],
        out_specs=pl.BlockSpec((_B, _V), lambda i: (0, 0)),
        scratch_shapes=[
            pltpu.VMEM((_B, _V), jnp.float32),
            pltpu.VMEM((_B, _V), jnp.bfloat16),
            pltpu.VMEM((_B, _C), jnp.bfloat16),
        ],
    )(syndrome, ht_f8, h_f8, channel_llrs, wcv, wvc, damp)
    return out


# speedup vs baseline: 3.0911x; 1.0647x over previous
import functools

import jax
import jax.numpy as jnp
from jax.experimental import pallas as pl
from jax.experimental.pallas import tpu as pltpu

_B = 64
_V = 4096
_C = 4096
_ITERS = 15
_TILE = 1024
_KT = _C // _TILE

_DIMS_NN = (((1,), (0,)), ((), ()))


def _bp_body(syn_ref, ht_ref, h_ref, llr_ref,
             wcv_ref, wvc_ref, damp_ref,
             out_ref, x_ref, xb_ref, cb_ref):
    i = pl.program_id(0)
    wcv = wcv_ref[0, 0]
    wvc = wvc_ref[0, 0]
    damp = damp_ref[0, 0]

    @pl.when(i == 0)
    def _():
        xb_ref[...] = llr_ref[...].astype(jnp.bfloat16)

    def ph0(t, carry):
        dst = pl.ds(t * _TILE, _TILE)
        v_to_c = wvc * jax.lax.dot_general(
            xb_ref[...], ht_ref[:, dst], _DIMS_NN,
            preferred_element_type=jnp.float32)
        s_sign = 1.0 - 2.0 * syn_ref[:, dst].astype(jnp.float32)
        c_msg = s_sign * jnp.tanh(v_to_c * 0.5)
        cb_ref[:, dst] = c_msg.astype(jnp.bfloat16)
        return carry

    jax.lax.fori_loop(0, _KT, ph0, 0, unroll=4)

    def ph1(t, carry):
        dst = pl.ds(t * _TILE, _TILE)
        c_to_v = wcv * jax.lax.dot_general(
            cb_ref[...], h_ref[:, dst], _DIMS_NN,
            preferred_element_type=jnp.float32)
        llr_t = llr_ref[:, dst]
        x_t = jnp.where(i == 0, llr_t, x_ref[:, dst])
        x_new = damp * x_t + (1.0 - damp) * (llr_t + c_to_v)
        x_ref[:, dst] = x_new
        xb_ref[:, dst] = x_new.astype(jnp.bfloat16)

        @pl.when(i == _ITERS - 1)
        def _():
            out_ref[:, dst] = jax.nn.sigmoid(-x_new)

        return carry

    jax.lax.fori_loop(0, _KT, ph1, 0, unroll=4)


@functools.partial(jax.jit, static_argnames=())
def kernel(syndrome, parity_matrix, channel_llrs, w_cv, w_vc, damping):
    h_f8 = parity_matrix.astype(jnp.float8_e4m3fn)
    ht_f8 = h_f8.T
    wcv = jnp.reshape(w_cv.astype(jnp.float32), (1, 1))
    wvc = jnp.reshape(w_vc.astype(jnp.float32), (1, 1))
    damp = jnp.reshape(damping.astype(jnp.float32), (1, 1))
    out = pl.pallas_call(
        _bp_body,
        grid=(_ITERS,),
        out_shape=jax.ShapeDtypeStruct((_B, _V), jnp.float32),
        in_specs=[
            pl.BlockSpec((_B, _C), lambda i: (0, 0)),
            pl.BlockSpec((_V, _C), lambda i: (0, 0)),
            pl.BlockSpec((_C, _V), lambda i: (0, 0)),
            pl.BlockSpec((_B, _V), lambda i: (0, 0)),
            pl.BlockSpec((1, 1), lambda i: (0, 0), memory_space=pltpu.SMEM),
            pl.BlockSpec((1, 1), lambda i: (0, 0), memory_space=pltpu.SMEM),
            pl.BlockSpec((1, 1), lambda i: (0, 0), memory_space=pltpu.SMEM),
        ],
        out_specs=pl.BlockSpec((_B, _V), lambda i: (0, 0)),
        scratch_shapes=[
            pltpu.VMEM((_B, _V), jnp.float32),
            pltpu.VMEM((_B, _V), jnp.bfloat16),
            pltpu.VMEM((_B, _C), jnp.bfloat16),
        ],
    )(syndrome, ht_f8, h_f8, channel_llrs, wcv, wvc, damp)
    return out
